# baseline (speedup 1.0000x reference)
import math
import functools

import jax
import jax.numpy as jnp
from jax import lax
from jax.experimental import pallas as pl
from jax.experimental.pallas import tpu as pltpu
from jax.experimental.pallas import tpu_sc as plsc

_N_USERS = 4000
_N_ITEMS = 6000
_N = _N_USERS + _N_ITEMS
_H = 128
_EG = 16
_E = 320000
_NPATH = 6

_NC = 2
_NS = 16
_NW = _NC * _NS
_EPW = _E // _NW
_C = 80
_NCH = _EPW // _C
_G = _C // 16
_RPT = _N // _NS
_RZ = 125
_HQ = _H // 4
_EPT_B = _E // _NS
_NCHB = _EPT_B // _C
_UR = 5
_UE = _UR * _C
_BR = 50
_UPB = _BR // _UR

_INV_SQRT_H = 1.0 / math.sqrt(float(_H))


def _mesh():
    return plsc.VectorSubcoreMesh(
        core_axis_name="c", subcore_axis_name="s",
        num_cores=_NC, num_subcores=_NS)


def _ln(x):
    mu = jnp.mean(x, axis=-1, keepdims=True)
    xc = x - mu
    var = jnp.mean(xc * xc, axis=-1, keepdims=True)
    return xc * lax.rsqrt(var + 1e-5)


def _prep1_body(emb_ref, eg_ref, lp_ref, a_ref, q0_ref, q1_ref, q2_ref, q3_ref, par_ref):
    y = _ln(emb_ref[...])
    a_ref[...] = jnp.concatenate((y, eg_ref[...]), axis=-1)
    q0_ref[...] = y[:, 0 * _HQ:1 * _HQ]
    q1_ref[...] = y[:, 1 * _HQ:2 * _HQ]
    q2_ref[...] = y[:, 2 * _HQ:3 * _HQ]
    q3_ref[...] = y[:, 3 * _HQ:4 * _HQ]
    par_ref[...] = jnp.exp(lp_ref[...])


def _prep2_body(plo_ref, phi_ref, eg_ref, lp_ref, emb_ref, a_ref, q0_ref,
                q1_ref, q2_ref, q3_ref, par_ref):
    emb = jnp.concatenate(
        (plo_ref[0], plo_ref[1], phi_ref[0], phi_ref[1]), axis=-1)
    emb_ref[...] = emb
    y = _ln(emb)
    a_ref[...] = jnp.concatenate((y, eg_ref[...]), axis=-1)
    q0_ref[...] = y[:, 0 * _HQ:1 * _HQ]
    q1_ref[...] = y[:, 1 * _HQ:2 * _HQ]
    q2_ref[...] = y[:, 2 * _HQ:3 * _HQ]
    q3_ref[...] = y[:, 3 * _HQ:4 * _HQ]
    par_ref[...] = jnp.exp(lp_ref[...])


def _final_body(e0_ref, e1_ref, plo_ref, phi_ref, out_ref):
    emb2 = jnp.concatenate(
        (plo_ref[0], plo_ref[1], phi_ref[0], phi_ref[1]), axis=-1)
    out_ref[...] = (e0_ref[...] + e1_ref[...] + emb2) * jnp.float32(1.0 / 3.0)


_RB = 1000


def _prep1(emb, eg, lp):
    rb = lambda i: (i, 0)
    z = lambda i: (0, 0)
    return pl.pallas_call(
        _prep1_body,
        grid=(_N // _RB,),
        in_specs=[pl.BlockSpec((_RB, _H), rb),
                  pl.BlockSpec((_RB, _EG), rb),
                  pl.BlockSpec((1, _H), z)],
        out_specs=[pl.BlockSpec((_RB, _H + _EG), rb)]
        + [pl.BlockSpec((_RB, _HQ), rb)] * 4
        + [pl.BlockSpec((1, _H), z)],
        out_shape=[jax.ShapeDtypeStruct((_N, _H + _EG), jnp.float32)]
        + [jax.ShapeDtypeStruct((_N, _HQ), jnp.float32)] * 4
        + [jax.ShapeDtypeStruct((1, _H), jnp.float32)],
    )(emb, eg, lp)


def _prep2(plo, phi, eg, lp):
    rb = lambda i: (i, 0)
    rb3 = lambda i: (0, i, 0)
    z = lambda i: (0, 0)
    return pl.pallas_call(
        _prep2_body,
        grid=(_N // _RB,),
        in_specs=[pl.BlockSpec((2, _RB, _HQ), rb3),
                  pl.BlockSpec((2, _RB, _HQ), rb3),
                  pl.BlockSpec((_RB, _EG), rb),
                  pl.BlockSpec((1, _H), z)],
        out_specs=[pl.BlockSpec((_RB, _H), rb),
                   pl.BlockSpec((_RB, _H + _EG), rb)]
        + [pl.BlockSpec((_RB, _HQ), rb)] * 4
        + [pl.BlockSpec((1, _H), z)],
        out_shape=[jax.ShapeDtypeStruct((_N, _H), jnp.float32),
                   jax.ShapeDtypeStruct((_N, _H + _EG), jnp.float32)]
        + [jax.ShapeDtypeStruct((_N, _HQ), jnp.float32)] * 4
        + [jax.ShapeDtypeStruct((1, _H), jnp.float32)],
    )(plo, phi, eg, lp)


def _final(emb0, emb1, plo, phi):
    return pl.pallas_call(
        _final_body,
        out_shape=jax.ShapeDtypeStruct((_N, _H), jnp.float32),
    )(emb0, emb1, plo, phi)


def _pass_a_body(a_h, i0_h, i01_h, pt_h, par_h,
                 ev0_h, dp_h,
                 i0a, i01a, pta, ev0a, ev1A, ev1B,
                 r01A, r01B, par_v, zb,
                 d0_sp, d1_sp, semA, semB, semSA, semSB):
    cid = lax.axis_index("c")
    sid = lax.axis_index("s")
    wid = cid * _NS + sid

    pltpu.sync_copy(par_h, par_v)

    zvec = jnp.zeros((16,), jnp.float32)

    def _zloop(i, _):
        zb[pl.ds(i * 16, 16)] = zvec
        return 0
    lax.fori_loop(0, _N // 16, _zloop, 0)

    @pl.when(sid == 0)
    def _():
        pltpu.sync_copy(zb, d0_sp)

    @pl.when(sid == 1)
    def _():
        pltpu.sync_copy(zb, d1_sp)

    plsc.subcore_barrier()

    c_lam = par_v[pl.ds(0, 16)][0]
    inv = jnp.float32(_INV_SQRT_H)
    lane = lax.iota(jnp.int32, 16)
    rowbase = wid * _NCH

    pltpu.sync_copy(i0_h.at[pl.ds(rowbase, _NCH), :], i0a)
    pltpu.sync_copy(i01_h.at[pl.ds(rowbase, _NCH), :], i01a)
    pltpu.sync_copy(pt_h.at[pl.ds(rowbase, _NCH), :], pta)

    def _fire(row, r01, sem):
        pltpu.async_copy(a_h.at[i01a.at[row]], r01, sem)

    def _drain(r01, sem):
        pltpu.make_async_copy(a_h.at[pl.ds(0, 2 * _C), :], r01, sem).wait()

    def _compute(row, r01, ev1buf, semS):
        def _group(gi, _):
            ev = jnp.zeros((16,), jnp.float32)
            for e in range(16):
                k = gi * 16 + e
                acc = r01[k, pl.ds(0, 16)] * r01[k + _C, pl.ds(0, 16)]
                for j in range(1, 8):
                    acc = acc + r01[k, pl.ds(j * 16, 16)] * r01[k + _C, pl.ds(j * 16, 16)]
                ed = r01[k, pl.ds(_H, 16)] * r01[k + _C, pl.ds(_H, 16)]
                v = acc * inv + ed * c_lam
                s = jnp.sum(v)
                ev = jnp.where(lane == e, s, ev)
            ev0a[row, pl.ds(gi * 16, 16)] = jnp.exp(ev)
            ptg = pta[row, pl.ds(gi * 16, 16)]
            ev1buf[pl.ds(gi * 16, 16)] = plsc.load_gather(par_v, [ptg + 1])
            return 0
        lax.fori_loop(0, _G, _group, 0)
        pltpu.async_copy(ev0a.at[row], d0_sp.at[i0a.at[row]], semS, add=True)
        pltpu.async_copy(ev1buf, d1_sp.at[i0a.at[row]], semS, add=True)

    def _drain_s(semS):
        pltpu.make_async_copy(ev0_h.at[0], ev1A, semS).wait()
        pltpu.make_async_copy(ev0_h.at[0], ev1A, semS).wait()

    _fire(0, r01A, semA)

    def _pair(k, _):
        r = 2 * k
        _fire(r + 1, r01B, semB)
        _drain(r01A, semA)

        @pl.when(k > 0)
        def _():
            _drain_s(semSA)
        _compute(r, r01A, ev1A, semSA)
        _fire(r + 2, r01A, semA)
        _drain(r01B, semB)

        @pl.when(k > 0)
        def _():
            _drain_s(semSB)
        _compute(r + 1, r01B, ev1B, semSB)
        return 0
    lax.fori_loop(0, (_NCH - 1) // 2, _pair, 0)

    _drain(r01A, semA)
    _drain_s(semSA)
    _compute(_NCH - 1, r01A, ev1A, semSA)
    _drain_s(semSA)
    _drain_s(semSB)

    pltpu.sync_copy(ev0a, ev0_h.at[pl.ds(rowbase, _NCH), :])

    plsc.subcore_barrier()

    @pl.when(sid == 0)
    def _():
        pltpu.sync_copy(d0_sp, zb)
        pltpu.sync_copy(zb, dp_h.at[cid, 0])

    @pl.when(sid == 1)
    def _():
        pltpu.sync_copy(d1_sp, zb)
        pltpu.sync_copy(zb, dp_h.at[cid, 1])


def _pass_a(a_tab, i0r, i01r, ptr, par):
    kfn = pl.kernel(
        _pass_a_body,
        out_type=(jax.ShapeDtypeStruct((_E // _C, _C), jnp.float32),
                  jax.ShapeDtypeStruct((_NC, 2, _N), jnp.float32)),
        mesh=_mesh(),
        compiler_params=pltpu.CompilerParams(needs_layout_passes=False, use_tc_tiling_on_sc=False),
        scratch_types=[
            pltpu.VMEM((_NCH, _C), jnp.int32),
            pltpu.VMEM((_NCH, 2 * _C), jnp.int32),
            pltpu.VMEM((_NCH, _C), jnp.int32),
            pltpu.VMEM((_NCH, _C), jnp.float32),
            pltpu.VMEM((_C,), jnp.float32),
            pltpu.VMEM((_C,), jnp.float32),
            pltpu.VMEM((2 * _C, _H + _EG), jnp.float32),
            pltpu.VMEM((2 * _C, _H + _EG), jnp.float32),
            pltpu.VMEM((16,), jnp.float32),
            pltpu.VMEM((_N,), jnp.float32),
            pltpu.VMEM_SHARED((_N,), jnp.float32),
            pltpu.VMEM_SHARED((_N,), jnp.float32),
            pltpu.SemaphoreType.DMA,
            pltpu.SemaphoreType.DMA,
            pltpu.SemaphoreType.DMA,
            pltpu.SemaphoreType.DMA,
        ],
    )
    return kfn(a_tab, i0r, i01r, ptr, par)


def _pass_b_body(ya_h, yb_h, i0b_h, i1_h, pt_h, par_h, ev_h, dp_h,
                 out_h,
                 i0uA, i0uB, i1uA, i1uB, i0a, pta, eva, rowsA, rowsB,
                 par_v, rd0, rd1, dtmp, tmp, out_sp,
                 semA, semB):
    cid = lax.axis_index("c")
    sid = lax.axis_index("s")

    pltpu.sync_copy(par_h, par_v)

    eps = jnp.float32(1e-16)
    pltpu.sync_copy(dp_h.at[0, 0], rd0)
    pltpu.sync_copy(dp_h.at[1, 0], dtmp)

    def _d0(i, _):
        sl = pl.ds(i * 16, 16)
        rd0[sl] = 1.0 / (rd0[sl] + dtmp[sl] + eps)
        return 0
    lax.fori_loop(0, _N // 16, _d0, 0)

    pltpu.sync_copy(dp_h.at[0, 1], rd1)
    pltpu.sync_copy(dp_h.at[1, 1], dtmp)

    def _d1(i, _):
        sl = pl.ds(i * 16, 16)
        rd1[sl] = 1.0 / (rd1[sl] + dtmp[sl] + eps)
        return 0
    lax.fori_loop(0, _N // 16, _d1, 0)

    zvec = jnp.zeros((16,), jnp.float32)

    def _z(i, _):
        for j in range(_HQ // 16):
            tmp[i, pl.ds(j * 16, 16)] = zvec
        return 0
    lax.fori_loop(0, _RZ, _z, 0)

    row0 = sid * _RPT
    for j in range(_RPT // _RZ):
        pltpu.sync_copy(tmp, out_sp.at[pl.ds(row0 + j * _RZ, _RZ), :])

    plsc.subcore_barrier()

    def _fire(ubase, u, i0u, i1u, rows, sem):
        ebase = (ubase + u) * _UE
        pltpu.sync_copy(i1_h.at[pl.ds(ebase, _UE)], i1u)

        @pl.when(cid == 0)
        def _():
            pltpu.async_copy(ya_h.at[i1u], rows, sem)

        @pl.when(cid == 1)
        def _():
            pltpu.async_copy(yb_h.at[i1u], rows, sem)

    def _drain(i1u, rows, sem):
        pltpu.make_async_copy(ya_h.at[i1u], rows, sem).wait()

    def _compute(u, i0u, rows):
        def _fill(j, _):
            sl = pl.ds(j * 16, 16)
            i0u[sl] = i0a[u, sl]
            return 0
        lax.fori_loop(0, _UE // 16, _fill, 0)

        def _group(gi, _):
            sl = pl.ds(gi * 16, 16)
            i0g = i0u[sl]
            d0 = plsc.load_gather(rd0, [i0g])
            d1 = plsc.load_gather(rd1, [i0g])
            ev0g = eva[u, sl]
            ptg = pta[u, sl]
            ev1g = plsc.load_gather(par_v, [ptg + 1])
            sg = jnp.float32(0.5) * (ev0g * d0 + ev1g * d1)
            for e in range(16):
                k = gi * 16 + e
                sv = sg[e]
                for j in range(_HQ // 16):
                    ksl = pl.ds(j * 16, 16)
                    rows[k, ksl] = rows[k, ksl] * sv
            return 0
        lax.fori_loop(0, _UR * _G, _group, 0)
        pltpu.sync_copy(rows, out_sp.at[i0u], add=True)

    def _block(b, _):
        ubase = sid * (_EPT_B // _UE) + b * _UPB
        pltpu.sync_copy(i0b_h.at[pl.ds(ubase, _UPB), :], i0a)
        pltpu.sync_copy(pt_h.at[pl.ds(ubase, _UPB), :], pta)
        pltpu.sync_copy(ev_h.at[pl.ds(ubase, _UPB), :], eva)

        _fire(ubase, 0, i0uA, i1uA, rowsA, semA)

        def _pairk(k, _):
            u = 2 * k
            _fire(ubase, u + 1, i0uB, i1uB, rowsB, semB)
            _drain(i1uA, rowsA, semA)
            _compute(u, i0uA, rowsA)
            _fire(ubase, u + 2, i0uA, i1uA, rowsA, semA)
            _drain(i1uB, rowsB, semB)
            _compute(u + 1, i0uB, rowsB)
            return 0
        lax.fori_loop(0, (_UPB - 2) // 2, _pairk, 0)

        _fire(ubase, _UPB - 1, i0uB, i1uB, rowsB, semB)
        _drain(i1uA, rowsA, semA)
        _compute(_UPB - 2, i0uA, rowsA)
        _drain(i1uB, rowsB, semB)
        _compute(_UPB - 1, i0uB, rowsB)
        return 0
    lax.fori_loop(0, _NCHB // _BR, _block, 0)

    plsc.subcore_barrier()

    for j in range(_RPT // _RZ):
        r = row0 + j * _RZ
        pltpu.sync_copy(out_sp.at[pl.ds(r, _RZ), :], tmp)
        pltpu.sync_copy(tmp, out_h.at[cid, pl.ds(r, _RZ), :])


def _pass_b(ya, yb, i0f, i1f, ptu, par, evu, dp):
    kfn = pl.kernel(
        _pass_b_body,
        out_type=jax.ShapeDtypeStruct((_NC, _N, _HQ), jnp.float32),
        mesh=_mesh(),
        compiler_params=pltpu.CompilerParams(needs_layout_passes=False, use_tc_tiling_on_sc=False),
        scratch_types=[
            pltpu.VMEM((_UE,), jnp.int32),
            pltpu.VMEM((_UE,), jnp.int32),
            pltpu.VMEM((_UE,), jnp.int32),
            pltpu.VMEM((_UE,), jnp.int32),
            pltpu.VMEM((_UPB, _UE), jnp.int32),
            pltpu.VMEM((_UPB, _UE), jnp.int32),
            pltpu.VMEM((_UPB, _UE), jnp.float32),
            pltpu.VMEM((_UE, _HQ), jnp.float32),
            pltpu.VMEM((_UE, _HQ), jnp.float32),
            pltpu.VMEM((16,), jnp.float32),
            pltpu.VMEM((_N,), jnp.float32),
            pltpu.VMEM((_N,), jnp.float32),
            pltpu.VMEM((_N,), jnp.float32),
            pltpu.VMEM((_RZ, _HQ), jnp.float32),
            pltpu.VMEM_SHARED((_N, _HQ), jnp.float32),
            pltpu.SemaphoreType.DMA,
            pltpu.SemaphoreType.DMA,
        ],
    )
    return kfn(ya, yb, i0f, i1f, ptu, par, evu, dp)


def _layer(a_tab, yq, par16, i0, i1, pt):
    shp = (_E // _C, _C)
    shpu = (_E // _UE, _UE)
    i0r = i0.reshape(shp)
    i1r = i1.reshape(shp)
    i01r = jnp.concatenate((i0r, i1r), axis=1)
    ptr = pt.reshape(shp)
    ev0r, dp = _pass_a(a_tab, i0r, i01r, ptr, par16)
    ptu = pt.reshape(shpu)
    evu = ev0r.reshape(shpu)
    i0b = i0.reshape(shpu)
    plo = _pass_b(yq[0], yq[1], i0b, i1, ptu, par16, evu, dp)
    phi = _pass_b(yq[2], yq[3], i0b, i1, ptu, par16, evu, dp)
    return plo, phi


def kernel(user_table, item_table, eigs, lambda0_0, path_w0, lambda0_1,
           path_w1, indices0, path_type0, indices1, path_type1):
    f32 = jnp.float32
    emb0 = jnp.concatenate([user_table, item_table], axis=0).astype(f32)
    eigs = eigs.astype(f32)
    i0_a = indices0[0].astype(jnp.int32)
    i1_a = indices0[1].astype(jnp.int32)
    pt_a = path_type0.astype(jnp.int32)
    i0_b = indices1[0].astype(jnp.int32)
    i1_b = indices1[1].astype(jnp.int32)
    pt_b = path_type1.astype(jnp.int32)

    def lp_of(lam, pw):
        return jnp.concatenate(
            [lam.reshape(-1).astype(f32), pw.reshape(-1).astype(f32),
             jnp.zeros((_H - 1 - _NPATH,), f32)]).reshape(1, _H)

    lp1 = lp_of(lambda0_0, path_w0)
    lp2 = lp_of(lambda0_1, path_w1)

    a1, q10, q11, q12, q13, par1 = _prep1(emb0, eigs, lp1)
    plo1, phi1 = _layer(a1, (q10, q11, q12, q13), par1.reshape(-1)[:16],
                        i0_a, i1_a, pt_a)
    emb1, a2, q20, q21, q22, q23, par2 = _prep2(plo1, phi1, eigs, lp2)
    plo2, phi2 = _layer(a2, (q20, q21, q22, q23), par2.reshape(-1)[:16],
                        i0_b, i1_b, pt_b)
    return _final(emb0, emb1, plo2, phi2)


# speedup vs baseline: 1.1373x; 1.1373x over previous
import math
import functools

import jax
import jax.numpy as jnp
from jax import lax
from jax.experimental import pallas as pl
from jax.experimental.pallas import tpu as pltpu
from jax.experimental.pallas import tpu_sc as plsc

_N_USERS = 4000
_N_ITEMS = 6000
_N = _N_USERS + _N_ITEMS
_H = 128
_EG = 16
_E = 320000
_NPATH = 6

_NC = 2
_NS = 16
_NW = _NC * _NS
_EPW = _E // _NW
_C = 80
_NCH = _EPW // _C
_G = _C // 16
_RPT = _N // _NS
_RZ = 125
_HQ = _H // 4
_EPT_B = _E // _NS
_NCHB = _EPT_B // _C
_UR = 5
_UE = _UR * _C
_BR = 50
_UPB = _BR // _UR

_INV_SQRT_H = 1.0 / math.sqrt(float(_H))


def _mesh():
    return plsc.VectorSubcoreMesh(
        core_axis_name="c", subcore_axis_name="s",
        num_cores=_NC, num_subcores=_NS)


def _ln(x):
    mu = jnp.mean(x, axis=-1, keepdims=True)
    xc = x - mu
    var = jnp.mean(xc * xc, axis=-1, keepdims=True)
    return xc * lax.rsqrt(var + 1e-5)


def _prep1_body(emb_ref, lp_ref, y_ref, q0_ref, q1_ref, q2_ref, q3_ref, par_ref):
    y = _ln(emb_ref[...])
    y_ref[...] = y
    q0_ref[...] = y[:, 0 * _HQ:1 * _HQ]
    q1_ref[...] = y[:, 1 * _HQ:2 * _HQ]
    q2_ref[...] = y[:, 2 * _HQ:3 * _HQ]
    q3_ref[...] = y[:, 3 * _HQ:4 * _HQ]
    par_ref[...] = jnp.exp(lp_ref[...])


def _prep2_body(plo_ref, phi_ref, lp_ref, emb_ref, y_ref, q0_ref, q1_ref,
                q2_ref, q3_ref, par_ref):
    emb = jnp.concatenate(
        (plo_ref[0], plo_ref[1], phi_ref[0], phi_ref[1]), axis=-1)
    emb_ref[...] = emb
    y = _ln(emb)
    y_ref[...] = y
    q0_ref[...] = y[:, 0 * _HQ:1 * _HQ]
    q1_ref[...] = y[:, 1 * _HQ:2 * _HQ]
    q2_ref[...] = y[:, 2 * _HQ:3 * _HQ]
    q3_ref[...] = y[:, 3 * _HQ:4 * _HQ]
    par_ref[...] = jnp.exp(lp_ref[...])


def _final_body(e0_ref, e1_ref, plo_ref, phi_ref, out_ref):
    emb2 = jnp.concatenate(
        (plo_ref[0], plo_ref[1], phi_ref[0], phi_ref[1]), axis=-1)
    out_ref[...] = (e0_ref[...] + e1_ref[...] + emb2) * jnp.float32(1.0 / 3.0)


def _prep1(emb, lp):
    return pl.pallas_call(
        _prep1_body,
        out_shape=(jax.ShapeDtypeStruct((_N, _H), jnp.float32),)
        + (jax.ShapeDtypeStruct((_N, _HQ), jnp.float32),) * 4
        + (jax.ShapeDtypeStruct((1, _H), jnp.float32),),
    )(emb, lp)


def _prep2(plo, phi, lp):
    return pl.pallas_call(
        _prep2_body,
        out_shape=(jax.ShapeDtypeStruct((_N, _H), jnp.float32),
                   jax.ShapeDtypeStruct((_N, _H), jnp.float32))
        + (jax.ShapeDtypeStruct((_N, _HQ), jnp.float32),) * 4
        + (jax.ShapeDtypeStruct((1, _H), jnp.float32),),
    )(plo, phi, lp)


def _final(emb0, emb1, plo, phi):
    return pl.pallas_call(
        _final_body,
        out_shape=jax.ShapeDtypeStruct((_N, _H), jnp.float32),
    )(emb0, emb1, plo, phi)


def _pass_a_body(y_h, eg_h, i0_h, i01_h, pt_h, par_h,
                 ev0_h, dp_h,
                 i0a, i01a, pta, ev0a, ev1A, ev1B,
                 r01A, e01A, r01B, e01B, par_v, zb,
                 d0_sp, d1_sp, semA, semB, semSA, semSB):
    cid = lax.axis_index("c")
    sid = lax.axis_index("s")
    wid = cid * _NS + sid

    pltpu.sync_copy(par_h, par_v)

    zvec = jnp.zeros((16,), jnp.float32)

    def _zloop(i, _):
        zb[pl.ds(i * 16, 16)] = zvec
        return 0
    lax.fori_loop(0, _N // 16, _zloop, 0)

    @pl.when(sid == 0)
    def _():
        pltpu.sync_copy(zb, d0_sp)

    @pl.when(sid == 1)
    def _():
        pltpu.sync_copy(zb, d1_sp)

    plsc.subcore_barrier()

    c_lam = par_v[pl.ds(0, 16)][0]
    inv = jnp.float32(_INV_SQRT_H)
    lane = lax.iota(jnp.int32, 16)
    rowbase = wid * _NCH

    pltpu.sync_copy(i0_h.at[pl.ds(rowbase, _NCH), :], i0a)
    pltpu.sync_copy(i01_h.at[pl.ds(rowbase, _NCH), :], i01a)
    pltpu.sync_copy(pt_h.at[pl.ds(rowbase, _NCH), :], pta)

    def _fire(row, r01, e01, sem):
        pltpu.async_copy(y_h.at[i01a.at[row]], r01, sem)
        pltpu.async_copy(eg_h.at[i01a.at[row]], e01, sem)

    def _drain(r01, e01, sem):
        pltpu.make_async_copy(y_h.at[pl.ds(0, 2 * _C), :], r01, sem).wait()
        pltpu.make_async_copy(eg_h.at[pl.ds(0, 2 * _C), :], e01, sem).wait()

    def _compute(row, r01, e01, ev1buf, semS):
        def _group(gi, _):
            ev = jnp.zeros((16,), jnp.float32)
            for e in range(16):
                k = gi * 16 + e
                acc = r01[k, pl.ds(0, 16)] * r01[k + _C, pl.ds(0, 16)]
                for j in range(1, 8):
                    acc = acc + r01[k, pl.ds(j * 16, 16)] * r01[k + _C, pl.ds(j * 16, 16)]
                ed = e01[k, :] * e01[k + _C, :]
                v = acc * inv + ed * c_lam
                s = jnp.sum(v)
                ev = jnp.where(lane == e, s, ev)
            ev0a[row, pl.ds(gi * 16, 16)] = jnp.exp(ev)
            ptg = pta[row, pl.ds(gi * 16, 16)]
            ev1buf[pl.ds(gi * 16, 16)] = plsc.load_gather(par_v, [ptg + 1])
            return 0
        lax.fori_loop(0, _G, _group, 0)
        pltpu.async_copy(ev0a.at[row], d0_sp.at[i0a.at[row]], semS, add=True)
        pltpu.async_copy(ev1buf, d1_sp.at[i0a.at[row]], semS, add=True)

    def _drain_s(semS):
        pltpu.make_async_copy(ev0_h.at[0], ev1A, semS).wait()
        pltpu.make_async_copy(ev0_h.at[0], ev1A, semS).wait()

    _fire(0, r01A, e01A, semA)

    def _pair(k, _):
        r = 2 * k
        _fire(r + 1, r01B, e01B, semB)
        _drain(r01A, e01A, semA)

        @pl.when(k > 0)
        def _():
            _drain_s(semSA)
        _compute(r, r01A, e01A, ev1A, semSA)
        _fire(r + 2, r01A, e01A, semA)
        _drain(r01B, e01B, semB)

        @pl.when(k > 0)
        def _():
            _drain_s(semSB)
        _compute(r + 1, r01B, e01B, ev1B, semSB)
        return 0
    lax.fori_loop(0, (_NCH - 1) // 2, _pair, 0)

    _drain(r01A, e01A, semA)
    _drain_s(semSA)
    _compute(_NCH - 1, r01A, e01A, ev1A, semSA)
    _drain_s(semSA)
    _drain_s(semSB)

    pltpu.sync_copy(ev0a, ev0_h.at[pl.ds(rowbase, _NCH), :])

    plsc.subcore_barrier()

    @pl.when(sid == 0)
    def _():
        pltpu.sync_copy(d0_sp, zb)
        pltpu.sync_copy(zb, dp_h.at[cid, 0])

    @pl.when(sid == 1)
    def _():
        pltpu.sync_copy(d1_sp, zb)
        pltpu.sync_copy(zb, dp_h.at[cid, 1])


def _pass_a(y, eigs, i0r, i01r, ptr, par):
    kfn = pl.kernel(
        _pass_a_body,
        out_type=(jax.ShapeDtypeStruct((_E // _C, _C), jnp.float32),
                  jax.ShapeDtypeStruct((_NC, 2, _N), jnp.float32)),
        mesh=_mesh(),
        compiler_params=pltpu.CompilerParams(needs_layout_passes=False, use_tc_tiling_on_sc=False),
        scratch_types=[
            pltpu.VMEM((_NCH, _C), jnp.int32),
            pltpu.VMEM((_NCH, 2 * _C), jnp.int32),
            pltpu.VMEM((_NCH, _C), jnp.int32),
            pltpu.VMEM((_NCH, _C), jnp.float32),
            pltpu.VMEM((_C,), jnp.float32),
            pltpu.VMEM((_C,), jnp.float32),
            pltpu.VMEM((2 * _C, _H), jnp.float32),
            pltpu.VMEM((2 * _C, _EG), jnp.float32),
            pltpu.VMEM((2 * _C, _H), jnp.float32),
            pltpu.VMEM((2 * _C, _EG), jnp.float32),
            pltpu.VMEM((16,), jnp.float32),
            pltpu.VMEM((_N,), jnp.float32),
            pltpu.VMEM_SHARED((_N,), jnp.float32),
            pltpu.VMEM_SHARED((_N,), jnp.float32),
            pltpu.SemaphoreType.DMA,
            pltpu.SemaphoreType.DMA,
            pltpu.SemaphoreType.DMA,
            pltpu.SemaphoreType.DMA,
        ],
    )
    return kfn(y, eigs, i0r, i01r, ptr, par)


def _pass_b_body(ya_h, yb_h, i0b_h, i1_h, pt_h, par_h, ev_h, dp_h,
                 out_h,
                 i0uA, i0uB, i1uA, i1uB, i0a, pta, eva, rowsA, rowsB,
                 par_v, rd0, rd1, dtmp, tmp, out_sp,
                 semA, semB):
    cid = lax.axis_index("c")
    sid = lax.axis_index("s")

    pltpu.sync_copy(par_h, par_v)

    eps = jnp.float32(1e-16)
    pltpu.sync_copy(dp_h.at[0, 0], rd0)
    pltpu.sync_copy(dp_h.at[1, 0], dtmp)

    def _d0(i, _):
        sl = pl.ds(i * 16, 16)
        rd0[sl] = 1.0 / (rd0[sl] + dtmp[sl] + eps)
        return 0
    lax.fori_loop(0, _N // 16, _d0, 0)

    pltpu.sync_copy(dp_h.at[0, 1], rd1)
    pltpu.sync_copy(dp_h.at[1, 1], dtmp)

    def _d1(i, _):
        sl = pl.ds(i * 16, 16)
        rd1[sl] = 1.0 / (rd1[sl] + dtmp[sl] + eps)
        return 0
    lax.fori_loop(0, _N // 16, _d1, 0)

    zvec = jnp.zeros((16,), jnp.float32)

    def _z(i, _):
        for j in range(_HQ // 16):
            tmp[i, pl.ds(j * 16, 16)] = zvec
        return 0
    lax.fori_loop(0, _RZ, _z, 0)

    row0 = sid * _RPT
    for j in range(_RPT // _RZ):
        pltpu.sync_copy(tmp, out_sp.at[pl.ds(row0 + j * _RZ, _RZ), :])

    plsc.subcore_barrier()

    def _fire(ubase, u, i0u, i1u, rows, sem):
        ebase = (ubase + u) * _UE
        pltpu.sync_copy(i1_h.at[pl.ds(ebase, _UE)], i1u)

        @pl.when(cid == 0)
        def _():
            pltpu.async_copy(ya_h.at[i1u], rows, sem)

        @pl.when(cid == 1)
        def _():
            pltpu.async_copy(yb_h.at[i1u], rows, sem)

    def _drain(i1u, rows, sem):
        pltpu.make_async_copy(ya_h.at[i1u], rows, sem).wait()

    def _compute(u, i0u, rows):
        def _fill(j, _):
            sl = pl.ds(j * 16, 16)
            i0u[sl] = i0a[u, sl]
            return 0
        lax.fori_loop(0, _UE // 16, _fill, 0)

        def _group(gi, _):
            sl = pl.ds(gi * 16, 16)
            i0g = i0u[sl]
            d0 = plsc.load_gather(rd0, [i0g])
            d1 = plsc.load_gather(rd1, [i0g])
            ev0g = eva[u, sl]
            ptg = pta[u, sl]
            ev1g = plsc.load_gather(par_v, [ptg + 1])
            sg = jnp.float32(0.5) * (ev0g * d0 + ev1g * d1)
            for e in range(16):
                k = gi * 16 + e
                sv = sg[e]
                for j in range(_HQ // 16):
                    ksl = pl.ds(j * 16, 16)
                    rows[k, ksl] = rows[k, ksl] * sv
            return 0
        lax.fori_loop(0, _UR * _G, _group, 0)
        pltpu.sync_copy(rows, out_sp.at[i0u], add=True)

    def _block(b, _):
        ubase = sid * (_EPT_B // _UE) + b * _UPB
        pltpu.sync_copy(i0b_h.at[pl.ds(ubase, _UPB), :], i0a)
        pltpu.sync_copy(pt_h.at[pl.ds(ubase, _UPB), :], pta)
        pltpu.sync_copy(ev_h.at[pl.ds(ubase, _UPB), :], eva)

        _fire(ubase, 0, i0uA, i1uA, rowsA, semA)

        def _pairk(k, _):
            u = 2 * k
            _fire(ubase, u + 1, i0uB, i1uB, rowsB, semB)
            _drain(i1uA, rowsA, semA)
            _compute(u, i0uA, rowsA)
            _fire(ubase, u + 2, i0uA, i1uA, rowsA, semA)
            _drain(i1uB, rowsB, semB)
            _compute(u + 1, i0uB, rowsB)
            return 0
        lax.fori_loop(0, (_UPB - 2) // 2, _pairk, 0)

        _fire(ubase, _UPB - 1, i0uB, i1uB, rowsB, semB)
        _drain(i1uA, rowsA, semA)
        _compute(_UPB - 2, i0uA, rowsA)
        _drain(i1uB, rowsB, semB)
        _compute(_UPB - 1, i0uB, rowsB)
        return 0
    lax.fori_loop(0, _NCHB // _BR, _block, 0)

    plsc.subcore_barrier()

    for j in range(_RPT // _RZ):
        r = row0 + j * _RZ
        pltpu.sync_copy(out_sp.at[pl.ds(r, _RZ), :], tmp)
        pltpu.sync_copy(tmp, out_h.at[cid, pl.ds(r, _RZ), :])


def _pass_b(ya, yb, i0f, i1f, ptu, par, evu, dp):
    kfn = pl.kernel(
        _pass_b_body,
        out_type=jax.ShapeDtypeStruct((_NC, _N, _HQ), jnp.float32),
        mesh=_mesh(),
        compiler_params=pltpu.CompilerParams(needs_layout_passes=False, use_tc_tiling_on_sc=False),
        scratch_types=[
            pltpu.VMEM((_UE,), jnp.int32),
            pltpu.VMEM((_UE,), jnp.int32),
            pltpu.VMEM((_UE,), jnp.int32),
            pltpu.VMEM((_UE,), jnp.int32),
            pltpu.VMEM((_UPB, _UE), jnp.int32),
            pltpu.VMEM((_UPB, _UE), jnp.int32),
            pltpu.VMEM((_UPB, _UE), jnp.float32),
            pltpu.VMEM((_UE, _HQ), jnp.float32),
            pltpu.VMEM((_UE, _HQ), jnp.float32),
            pltpu.VMEM((16,), jnp.float32),
            pltpu.VMEM((_N,), jnp.float32),
            pltpu.VMEM((_N,), jnp.float32),
            pltpu.VMEM((_N,), jnp.float32),
            pltpu.VMEM((_RZ, _HQ), jnp.float32),
            pltpu.VMEM_SHARED((_N, _HQ), jnp.float32),
            pltpu.SemaphoreType.DMA,
            pltpu.SemaphoreType.DMA,
        ],
    )
    return kfn(ya, yb, i0f, i1f, ptu, par, evu, dp)


def _layer(y, yq, par16, eigs, i0, i1, pt):
    shp = (_E // _C, _C)
    shpu = (_E // _UE, _UE)
    i0r = i0.reshape(shp)
    i1r = i1.reshape(shp)
    i01r = jnp.concatenate((i0r, i1r), axis=1)
    ptr = pt.reshape(shp)
    ev0r, dp = _pass_a(y, eigs, i0r, i01r, ptr, par16)
    ptu = pt.reshape(shpu)
    evu = ev0r.reshape(shpu)
    i0b = i0.reshape(shpu)
    plo = _pass_b(yq[0], yq[1], i0b, i1, ptu, par16, evu, dp)
    phi = _pass_b(yq[2], yq[3], i0b, i1, ptu, par16, evu, dp)
    return plo, phi


def kernel(user_table, item_table, eigs, lambda0_0, path_w0, lambda0_1,
           path_w1, indices0, path_type0, indices1, path_type1):
    f32 = jnp.float32
    emb0 = jnp.concatenate([user_table, item_table], axis=0).astype(f32)
    eigs = eigs.astype(f32)
    i0_a = indices0[0].astype(jnp.int32)
    i1_a = indices0[1].astype(jnp.int32)
    pt_a = path_type0.astype(jnp.int32)
    i0_b = indices1[0].astype(jnp.int32)
    i1_b = indices1[1].astype(jnp.int32)
    pt_b = path_type1.astype(jnp.int32)

    def lp_of(lam, pw):
        return jnp.concatenate(
            [lam.reshape(-1).astype(f32), pw.reshape(-1).astype(f32),
             jnp.zeros((_H - 1 - _NPATH,), f32)]).reshape(1, _H)

    lp1 = lp_of(lambda0_0, path_w0)
    lp2 = lp_of(lambda0_1, path_w1)

    y1, q10, q11, q12, q13, par1 = _prep1(emb0, lp1)
    plo1, phi1 = _layer(y1, (q10, q11, q12, q13), par1.reshape(-1)[:16],
                        eigs, i0_a, i1_a, pt_a)
    emb1, y2, q20, q21, q22, q23, par2 = _prep2(plo1, phi1, lp2)
    plo2, phi2 = _layer(y2, (q20, q21, q22, q23), par2.reshape(-1)[:16],
                        eigs, i0_b, i1_b, pt_b)
    return _final(emb0, emb1, plo2, phi2)
